# baseline (speedup 1.0000x reference)
import jax
import jax.numpy as jnp
from jax.experimental import pallas as pl
from jax.experimental.pallas import tpu as pltpu

H, W, P, D, EMB = 512, 512, 16, 128, 768
NR = H // P
NC = W // P
N_PATCH = NR * NC
BB = 4
NPB = N_PATCH // 2


def _add_kernel(x_ref, row_ref, col_ref, out_ref, pos_ref):
    @pl.when((pl.program_id(0) == 0) & (pl.program_id(1) == 0))
    def _build():
        r = row_ref[...].reshape(NR, D // NR, EMB)[:, 2, :]
        c = col_ref[...].reshape(NC, D // NC, EMB)[:, 2, :]
        pos_ref[...] = (r[:, None, :] + c[None, :, :]).reshape(N_PATCH, EMB)

    j = pl.program_id(1)
    out_ref[...] = x_ref[...] + pos_ref[pl.ds(j * NPB, NPB), :][None, :, :]


@jax.jit
def kernel(inputs, row_embedding, col_embedding):
    B = inputs.shape[0]
    grid = (B // BB, N_PATCH // NPB)
    return pl.pallas_call(
        _add_kernel,
        grid=grid,
        in_specs=[
            pl.BlockSpec((BB, NPB, EMB), lambda i, j: (i, j, 0)),
            pl.BlockSpec((D, EMB), lambda i, j: (0, 0)),
            pl.BlockSpec((D, EMB), lambda i, j: (0, 0)),
        ],
        out_specs=pl.BlockSpec((BB, NPB, EMB), lambda i, j: (i, j, 0)),
        out_shape=jax.ShapeDtypeStruct(inputs.shape, inputs.dtype),
        scratch_shapes=[pltpu.VMEM((N_PATCH, EMB), jnp.float32)],
    )(inputs, row_embedding, col_embedding)
